# Initial kernel scaffold; baseline (speedup 1.0000x reference)
import jax
import jax.numpy as jnp
from jax.experimental import pallas as pl


def kernel(x, edge_index, W1, b1, W2, b2):
    raise NotImplementedError("write your pallas kernel here")


# speedup vs baseline: 14.4965x; 14.4965x over previous
import functools

import jax
import jax.numpy as jnp
from jax import lax
from jax.experimental import pallas as pl
from jax.experimental.pallas import tpu as pltpu
from jax.experimental.pallas import tpu_sc as plsc

N = 10000
E = 320000
D_IN = 128
D_HID = 128
N_CLS = 40

NP = 10240
EP = 327680
NC = 2
NS = 16
NW = NC * NS
K = 128
NCH = EP // NW // K
RPT = NP // NS
D2 = 48

@functools.lru_cache(maxsize=None)
def _build_deg_kernel():
    mesh = plsc.VectorSubcoreMesh(core_axis_name="c", subcore_axis_name="s")
    return functools.partial(
        pl.kernel,
        out_type=jax.ShapeDtypeStruct((NC, NP), jnp.float32),
        mesh=mesh,
        scratch_types=[
            pltpu.VMEM((K,), jnp.float32),
            pltpu.VMEM((RPT,), jnp.float32),
            pltpu.VMEM((NCH, K), jnp.int32),
            pltpu.VMEM_SHARED((NP,), jnp.float32),
        ],
    )(_deg_body)


def _deg_body(dst_hbm, out_hbm, ones_v, zst_v, dst_v, deg_sh):
    c = lax.axis_index("c")
    s = lax.axis_index("s")
    wid = c * NS + s

    def _fill16(i, _):
        ones_v[pl.ds(i * 16, 16)] = jnp.ones((16,), jnp.float32)
        return 0

    lax.fori_loop(0, K // 16, _fill16, 0)

    def _zfill(i, _):
        zst_v[pl.ds(i * 16, 16)] = jnp.zeros((16,), jnp.float32)
        return 0

    lax.fori_loop(0, RPT // 16, _zfill, 0)
    pltpu.sync_copy(zst_v, deg_sh.at[pl.ds(s * RPT, RPT)])
    plsc.subcore_barrier()

    pltpu.sync_copy(dst_hbm.at[wid], dst_v)

    def _chunk(j, _):
        pltpu.sync_copy(ones_v, deg_sh.at[dst_v.at[j]], add=True)
        return 0

    lax.fori_loop(0, NCH, _chunk, 0)
    plsc.subcore_barrier()
    pltpu.sync_copy(deg_sh.at[pl.ds(s * RPT, RPT)],
                    out_hbm.at[c, pl.ds(s * RPT, RPT)])


@functools.lru_cache(maxsize=None)
def _make_edge_scatter(D):
    mesh = plsc.VectorSubcoreMesh(core_axis_name="c", subcore_axis_name="s")

    IB = 16

    @functools.partial(
        pl.kernel,
        out_type=jax.ShapeDtypeStruct((NC, NP, D), jnp.float32),
        mesh=mesh,
        scratch_types=[
            pltpu.VMEM((K, D), jnp.float32),
            pltpu.VMEM((K, D), jnp.float32),
            pltpu.VMEM((IB, K), jnp.int32),
            pltpu.VMEM((IB, K), jnp.int32),
            pltpu.VMEM_SHARED((NP, D), jnp.float32),
            pltpu.SemaphoreType.DMA,
            pltpu.SemaphoreType.DMA,
        ],
        compiler_params=pltpu.CompilerParams(use_tc_tiling_on_sc=False),
    )
    def _scatter(g_hbm, src_hbm, dst_hbm, out_hbm,
                 rows0, rows1, src_v, dst_v, acc_sh, sem0, sem1):
        c = lax.axis_index("c")
        s = lax.axis_index("s")
        wid = c * NS + s
        rows = (rows0, rows1)
        sems = (sem0, sem1)

        def _zrow(r, _):
            for j in range(D // 16):
                rows0[r, pl.ds(j * 16, 16)] = jnp.zeros((16,), jnp.float32)
            return 0

        lax.fori_loop(0, K, _zrow, 0)

        def _zcp(i, _):
            pltpu.sync_copy(rows0, acc_sh.at[pl.ds(s * RPT + i * K, K)])
            return 0

        lax.fori_loop(0, RPT // K, _zcp, 0)
        plsc.subcore_barrier()

        def _start(j, b):
            pltpu.async_copy(g_hbm.at[src_v.at[j]], rows[b], sems[b])

        def _wait(j, b):
            pltpu.make_async_copy(g_hbm.at[src_v.at[j]], rows[b],
                                  sems[b]).wait()

        def _block(ib, _):
            pltpu.sync_copy(src_hbm.at[wid, pl.ds(ib * IB, IB)], src_v)
            pltpu.sync_copy(dst_hbm.at[wid, pl.ds(ib * IB, IB)], dst_v)
            _start(0, 0)
            _start(1, 1)

            def _pair(j2, _):
                for b in range(2):
                    j = j2 * 2 + b
                    _wait(j, b)
                    pltpu.sync_copy(rows[b], acc_sh.at[dst_v.at[j]],
                                    add=True)

                    @pl.when(j + 2 < IB)
                    def _():
                        _start(j + 2, b)

                return 0

            lax.fori_loop(0, IB // 2, _pair, 0)
            return 0

        lax.fori_loop(0, NCH // IB, _block, 0)
        plsc.subcore_barrier()
        pltpu.sync_copy(acc_sh.at[pl.ds(s * RPT, RPT)],
                        out_hbm.at[c, pl.ds(s * RPT, RPT)])

    return _scatter


def _tc1_body(deg_ref, x_ref, w_ref, g_ref, dis_ref):
    d = deg_ref[0] + deg_ref[1] + 1.0
    dis = lax.rsqrt(d)
    h = jnp.dot(x_ref[...], w_ref[...], preferred_element_type=jnp.float32)
    g_ref[...] = h * dis
    dis_ref[...] = dis


def _tc2_body(s_ref, g_ref, dis_ref, b_ref, w_ref, out_ref):
    dis = dis_ref[...]
    pre = dis * (s_ref[0] + s_ref[1] + g_ref[...]) + b_ref[...]
    h = jnp.maximum(pre, 0.0)
    out_ref[...] = jnp.dot(h, w_ref[...],
                           preferred_element_type=jnp.float32) * dis


def _tc3_body(s_ref, g_ref, dis_ref, b_ref, out_ref):
    pre = dis_ref[...] * (s_ref[0] + s_ref[1] + g_ref[...]) + b_ref[...]
    z = pre[:, :N_CLS]
    m = jnp.max(z, axis=1, keepdims=True)
    ez = jnp.exp(z - m)
    lse = jnp.log(jnp.sum(ez, axis=1, keepdims=True))
    out_ref[...] = z - m - lse


def _f32(*shape):
    return jax.ShapeDtypeStruct(shape, jnp.float32)


def kernel(x, edge_index, W1, b1, W2, b2):
    ei = edge_index.astype(jnp.int32)
    pad = jnp.full((EP - E,), NP - 1, jnp.int32)
    src3 = jnp.concatenate([ei[0], pad]).reshape(NW, NCH, K)
    dst3 = jnp.concatenate([ei[1], pad]).reshape(NW, NCH, K)
    x_p = jnp.pad(x, ((0, NP - N), (0, 0)))
    b1r = b1.reshape(1, D_HID)
    W2p = jnp.pad(W2, ((0, 0), (0, D2 - N_CLS)))
    b2r = jnp.pad(b2, (0, D2 - N_CLS)).reshape(1, D2)

    deg2 = _build_deg_kernel()(dst3)
    deg3 = deg2.reshape(NC, NP, 1)

    g1, dis = pl.pallas_call(
        _tc1_body,
        out_shape=(_f32(NP, D_HID), _f32(NP, 1)),
    )(deg3, x_p, W1)

    s1 = _make_edge_scatter(D_HID)(g1, src3, dst3)

    g2 = pl.pallas_call(
        _tc2_body,
        out_shape=_f32(NP, D2),
    )(s1, g1, dis, b1r, W2p)

    s2 = _make_edge_scatter(D2)(g2, src3, dst3)

    out = pl.pallas_call(
        _tc3_body,
        out_shape=_f32(NP, N_CLS),
    )(s2, g2, dis, b2r)

    return out[:N]


# speedup vs baseline: 14.9396x; 1.0306x over previous
import functools

import jax
import jax.numpy as jnp
from jax import lax
from jax.experimental import pallas as pl
from jax.experimental.pallas import tpu as pltpu
from jax.experimental.pallas import tpu_sc as plsc

N = 10000
E = 320000
D_IN = 128
D_HID = 128
N_CLS = 40

NP = 10240
EP = 327680
NC = 2
NS = 16
NW = NC * NS
K = 128
NCH = EP // NW // K
RPT = NP // NS
D2 = 48

@functools.lru_cache(maxsize=None)
def _build_deg_kernel():
    mesh = plsc.VectorSubcoreMesh(core_axis_name="c", subcore_axis_name="s")
    return functools.partial(
        pl.kernel,
        out_type=jax.ShapeDtypeStruct((NC, NP), jnp.float32),
        mesh=mesh,
        scratch_types=[
            pltpu.VMEM((K,), jnp.float32),
            pltpu.VMEM((RPT,), jnp.float32),
            pltpu.VMEM((NCH, K), jnp.int32),
            pltpu.VMEM_SHARED((NP,), jnp.float32),
        ],
    )(_deg_body)


def _deg_body(dst_hbm, out_hbm, ones_v, zst_v, dst_v, deg_sh):
    c = lax.axis_index("c")
    s = lax.axis_index("s")
    wid = c * NS + s

    def _fill16(i, _):
        ones_v[pl.ds(i * 16, 16)] = jnp.ones((16,), jnp.float32)
        return 0

    lax.fori_loop(0, K // 16, _fill16, 0)

    def _zfill(i, _):
        zst_v[pl.ds(i * 16, 16)] = jnp.zeros((16,), jnp.float32)
        return 0

    lax.fori_loop(0, RPT // 16, _zfill, 0)
    pltpu.sync_copy(zst_v, deg_sh.at[pl.ds(s * RPT, RPT)])
    plsc.subcore_barrier()

    pltpu.sync_copy(dst_hbm.at[wid], dst_v)

    def _chunk(j, _):
        pltpu.sync_copy(ones_v, deg_sh.at[dst_v.at[j]], add=True)
        return 0

    lax.fori_loop(0, NCH, _chunk, 0)
    plsc.subcore_barrier()
    pltpu.sync_copy(deg_sh.at[pl.ds(s * RPT, RPT)],
                    out_hbm.at[c, pl.ds(s * RPT, RPT)])


@functools.lru_cache(maxsize=None)
def _make_edge_scatter(D):
    mesh = plsc.VectorSubcoreMesh(core_axis_name="c", subcore_axis_name="s")

    IB = 16

    @functools.partial(
        pl.kernel,
        out_type=jax.ShapeDtypeStruct((NC, NP, D), jnp.float32),
        mesh=mesh,
        scratch_types=[
            pltpu.VMEM((K, D), jnp.float32),
            pltpu.VMEM((K, D), jnp.float32),
            pltpu.VMEM((IB, K), jnp.int32),
            pltpu.VMEM((IB, K), jnp.int32),
            pltpu.VMEM_SHARED((NP, D), jnp.float32),
            pltpu.SemaphoreType.DMA,
            pltpu.SemaphoreType.DMA,
        ],
        compiler_params=pltpu.CompilerParams(use_tc_tiling_on_sc=False),
    )
    def _scatter(g_hbm, src_hbm, dst_hbm, out_hbm,
                 rows0, rows1, src_v, dst_v, acc_sh, sem0, sem1):
        c = lax.axis_index("c")
        s = lax.axis_index("s")
        wid = c * NS + s
        rows = (rows0, rows1)
        sems = (sem0, sem1)

        def _zrow(r, _):
            for j in range(D // 16):
                rows0[r, pl.ds(j * 16, 16)] = jnp.zeros((16,), jnp.float32)
            return 0

        lax.fori_loop(0, K, _zrow, 0)

        def _zcp(i, _):
            pltpu.sync_copy(rows0, acc_sh.at[pl.ds(s * RPT + i * K, K)])
            return 0

        lax.fori_loop(0, RPT // K, _zcp, 0)
        plsc.subcore_barrier()

        def _start(j, b):
            pltpu.async_copy(g_hbm.at[src_v.at[j]], rows[b], sems[b])

        def _wait(j, b):
            pltpu.make_async_copy(g_hbm.at[src_v.at[j]], rows[b],
                                  sems[b]).wait()

        def _block(ib, _):
            pltpu.sync_copy(src_hbm.at[wid, pl.ds(ib * IB, IB)], src_v)
            pltpu.sync_copy(dst_hbm.at[wid, pl.ds(ib * IB, IB)], dst_v)
            _start(0, 0)
            _start(1, 1)

            def _pair(j2, _):
                for b in range(2):
                    j = j2 * 2 + b
                    _wait(j, b)
                    pltpu.sync_copy(rows[b], acc_sh.at[dst_v.at[j]],
                                    add=True)

                    @pl.when(j + 2 < IB)
                    def _():
                        _start(j + 2, b)

                return 0

            lax.fori_loop(0, IB // 2, _pair, 0)
            return 0

        lax.fori_loop(0, NCH // IB, _block, 0)
        plsc.subcore_barrier()
        pltpu.sync_copy(acc_sh.at[pl.ds(s * RPT, RPT)],
                        out_hbm.at[c, pl.ds(s * RPT, RPT)])

    return _scatter


def _tc1_body(deg_ref, x_ref, w_ref, g_ref, dis_ref):
    d = deg_ref[0] + deg_ref[1] + 1.0
    dis = lax.rsqrt(d)
    h = jnp.dot(x_ref[...], w_ref[...], preferred_element_type=jnp.float32)
    g_ref[...] = h * dis
    dis_ref[...] = dis


def _tc2_body(s_ref, g_ref, dis_ref, b_ref, w_ref, out_ref):
    dis = dis_ref[...]
    pre = dis * (s_ref[0] + s_ref[1] + g_ref[...]) + b_ref[...]
    h = jnp.maximum(pre, 0.0)
    out_ref[...] = jnp.dot(h, w_ref[...],
                           preferred_element_type=jnp.float32) * dis


def _tc3_body(s_ref, g_ref, dis_ref, b_ref, out_ref):
    pre = dis_ref[...] * (s_ref[0] + s_ref[1] + g_ref[...]) + b_ref[...]
    z = pre[:, :N_CLS]
    m = jnp.max(z, axis=1, keepdims=True)
    ez = jnp.exp(z - m)
    lse = jnp.log(jnp.sum(ez, axis=1, keepdims=True))
    out_ref[...] = z - m - lse


def _f32(*shape):
    return jax.ShapeDtypeStruct(shape, jnp.float32)


def kernel(x, edge_index, W1, b1, W2, b2):
    ei = edge_index.astype(jnp.int32)
    src_pad = jnp.full((EP - E,), NP - 1, jnp.int32)
    dst_pad = N + (jnp.arange(EP - E, dtype=jnp.int32) % (NP - N))
    src3 = jnp.concatenate([ei[0], src_pad]).reshape(NW, NCH, K)
    dst3 = jnp.concatenate([ei[1], dst_pad]).reshape(NW, NCH, K)
    x_p = jnp.pad(x, ((0, NP - N), (0, 0)))
    b1r = b1.reshape(1, D_HID)
    W2p = jnp.pad(W2, ((0, 0), (0, D2 - N_CLS)))
    b2r = jnp.pad(b2, (0, D2 - N_CLS)).reshape(1, D2)

    deg2 = _build_deg_kernel()(dst3)
    deg3 = deg2.reshape(NC, NP, 1)

    g1, dis = pl.pallas_call(
        _tc1_body,
        out_shape=(_f32(NP, D_HID), _f32(NP, 1)),
    )(deg3, x_p, W1)

    s1 = _make_edge_scatter(D_HID)(g1, src3, dst3)

    g2 = pl.pallas_call(
        _tc2_body,
        out_shape=_f32(NP, D2),
    )(s1, g1, dis, b1r, W2p)

    s2 = _make_edge_scatter(D2)(g2, src3, dst3)

    out = pl.pallas_call(
        _tc3_body,
        out_shape=_f32(NP, N_CLS),
    )(s2, g2, dis, b2r)

    return out[:N]


# speedup vs baseline: 15.1028x; 1.0109x over previous
import functools

import jax
import jax.numpy as jnp
from jax import lax
from jax.experimental import pallas as pl
from jax.experimental.pallas import tpu as pltpu
from jax.experimental.pallas import tpu_sc as plsc

N = 10000
E = 320000
D_IN = 128
D_HID = 128
N_CLS = 40

NP = 10240
EP = 327680
NC = 2
NS = 16
NW = NC * NS
K = 128
NCH = EP // NW // K
KS = 64
NCHS = EP // NW // KS
RPT = NP // NS
D2 = 48

@functools.lru_cache(maxsize=None)
def _build_deg_kernel():
    mesh = plsc.VectorSubcoreMesh(core_axis_name="c", subcore_axis_name="s")
    return functools.partial(
        pl.kernel,
        out_type=jax.ShapeDtypeStruct((NC, NP), jnp.float32),
        mesh=mesh,
        scratch_types=[
            pltpu.VMEM((K,), jnp.float32),
            pltpu.VMEM((RPT,), jnp.float32),
            pltpu.VMEM((NCH, K), jnp.int32),
            pltpu.VMEM_SHARED((NP,), jnp.float32),
        ],
    )(_deg_body)


def _deg_body(dst_hbm, out_hbm, ones_v, zst_v, dst_v, deg_sh):
    c = lax.axis_index("c")
    s = lax.axis_index("s")
    wid = c * NS + s

    def _fill16(i, _):
        ones_v[pl.ds(i * 16, 16)] = jnp.ones((16,), jnp.float32)
        return 0

    lax.fori_loop(0, K // 16, _fill16, 0)

    def _zfill(i, _):
        zst_v[pl.ds(i * 16, 16)] = jnp.zeros((16,), jnp.float32)
        return 0

    lax.fori_loop(0, RPT // 16, _zfill, 0)
    pltpu.sync_copy(zst_v, deg_sh.at[pl.ds(s * RPT, RPT)])
    plsc.subcore_barrier()

    pltpu.sync_copy(dst_hbm.at[wid], dst_v)

    def _chunk(j, _):
        pltpu.sync_copy(ones_v, deg_sh.at[dst_v.at[j]], add=True)
        return 0

    lax.fori_loop(0, NCH, _chunk, 0)
    plsc.subcore_barrier()
    pltpu.sync_copy(deg_sh.at[pl.ds(s * RPT, RPT)],
                    out_hbm.at[c, pl.ds(s * RPT, RPT)])


@functools.lru_cache(maxsize=None)
def _make_edge_scatter(D):
    mesh = plsc.VectorSubcoreMesh(core_axis_name="c", subcore_axis_name="s")

    NB = 4
    IB = 32

    @functools.partial(
        pl.kernel,
        out_type=jax.ShapeDtypeStruct((NC, NP, D), jnp.float32),
        mesh=mesh,
        scratch_types=[
            [pltpu.VMEM((KS, D), jnp.float32) for _ in range(NB)],
            pltpu.VMEM((IB, KS), jnp.int32),
            pltpu.VMEM((IB, KS), jnp.int32),
            pltpu.VMEM_SHARED((NP, D), jnp.float32),
            [pltpu.SemaphoreType.DMA for _ in range(NB)],
            [pltpu.SemaphoreType.DMA for _ in range(NB)],
        ],
        compiler_params=pltpu.CompilerParams(use_tc_tiling_on_sc=False),
    )
    def _scatter(g_hbm, src_hbm, dst_hbm, out_hbm,
                 rows, src_v, dst_v, acc_sh, gsems, ssems):
        c = lax.axis_index("c")
        s = lax.axis_index("s")
        wid = c * NS + s

        def _zrow(r, _):
            for b in range(NB):
                for j in range(D // 16):
                    rows[b][r, pl.ds(j * 16, 16)] = jnp.zeros((16,),
                                                              jnp.float32)
            return 0

        lax.fori_loop(0, KS, _zrow, 0)

        def _zcp(i, _):
            pltpu.sync_copy(rows[0],
                            acc_sh.at[pl.ds(s * RPT + i * KS, KS)])
            return 0

        lax.fori_loop(0, RPT // KS, _zcp, 0)
        plsc.subcore_barrier()

        def _startg(j, b):
            pltpu.async_copy(g_hbm.at[src_v.at[j]], rows[b], gsems[b])

        def _waitg(j, b):
            pltpu.make_async_copy(g_hbm.at[src_v.at[j]], rows[b],
                                  gsems[b]).wait()

        def _starts(j, b):
            pltpu.async_copy(rows[b], acc_sh.at[dst_v.at[j]], ssems[b],
                             add=True)

        def _waits(j, b):
            pltpu.make_async_copy(rows[b], acc_sh.at[dst_v.at[j]],
                                  ssems[b]).wait()

        def _block(ib, _):
            pltpu.sync_copy(src_hbm.at[wid, pl.ds(ib * IB, IB)], src_v)
            pltpu.sync_copy(dst_hbm.at[wid, pl.ds(ib * IB, IB)], dst_v)
            _startg(0, 0)
            _startg(1, 1)
            _startg(2, 2)

            def _quad(j4, _):
                for b in range(NB):
                    j = j4 * NB + b
                    _waitg(j, b)
                    _starts(j, b)
                    bn = (b + NB - 1) % NB

                    @pl.when(j + NB - 1 < IB)
                    def _():
                        @pl.when(j > 0)
                        def _():
                            _waits(j - 1, bn)

                        _startg(j + NB - 1, bn)

                return 0

            lax.fori_loop(0, IB // NB, _quad, 0)
            for b in range(NB):
                _waits(IB - NB + b, b)
            return 0

        lax.fori_loop(0, NCHS // IB, _block, 0)
        plsc.subcore_barrier()
        pltpu.sync_copy(acc_sh.at[pl.ds(s * RPT, RPT)],
                        out_hbm.at[c, pl.ds(s * RPT, RPT)])

    return _scatter


def _tc1_body(deg_ref, x_ref, w_ref, g_ref, dis_ref):
    d = deg_ref[0] + deg_ref[1] + 1.0
    dis = lax.rsqrt(d)
    h = jnp.dot(x_ref[...], w_ref[...], preferred_element_type=jnp.float32)
    g_ref[...] = h * dis
    dis_ref[...] = dis


def _tc2_body(s_ref, g_ref, dis_ref, b_ref, w_ref, out_ref):
    dis = dis_ref[...]
    pre = dis * (s_ref[0] + s_ref[1] + g_ref[...]) + b_ref[...]
    h = jnp.maximum(pre, 0.0)
    out_ref[...] = jnp.dot(h, w_ref[...],
                           preferred_element_type=jnp.float32) * dis


def _tc3_body(s_ref, g_ref, dis_ref, b_ref, out_ref):
    pre = dis_ref[...] * (s_ref[0] + s_ref[1] + g_ref[...]) + b_ref[...]
    z = pre[:, :N_CLS]
    m = jnp.max(z, axis=1, keepdims=True)
    ez = jnp.exp(z - m)
    lse = jnp.log(jnp.sum(ez, axis=1, keepdims=True))
    out_ref[...] = z - m - lse


def _f32(*shape):
    return jax.ShapeDtypeStruct(shape, jnp.float32)


def kernel(x, edge_index, W1, b1, W2, b2):
    ei = edge_index.astype(jnp.int32)
    src_pad = jnp.full((EP - E,), NP - 1, jnp.int32)
    dst_pad = N + (jnp.arange(EP - E, dtype=jnp.int32) % (NP - N))
    src3 = jnp.concatenate([ei[0], src_pad]).reshape(NW, NCH, K)
    dst3 = jnp.concatenate([ei[1], dst_pad]).reshape(NW, NCH, K)
    x_p = jnp.pad(x, ((0, NP - N), (0, 0)))
    b1r = b1.reshape(1, D_HID)
    W2p = jnp.pad(W2, ((0, 0), (0, D2 - N_CLS)))
    b2r = jnp.pad(b2, (0, D2 - N_CLS)).reshape(1, D2)

    src3s = src3.reshape(NW, NCHS, KS)
    dst3s = dst3.reshape(NW, NCHS, KS)

    deg2 = _build_deg_kernel()(dst3)
    deg3 = deg2.reshape(NC, NP, 1)

    g1, dis = pl.pallas_call(
        _tc1_body,
        out_shape=(_f32(NP, D_HID), _f32(NP, 1)),
    )(deg3, x_p, W1)

    s1 = _make_edge_scatter(D_HID)(g1, src3s, dst3s)

    g2 = pl.pallas_call(
        _tc2_body,
        out_shape=_f32(NP, D2),
    )(s1, g1, dis, b1r, W2p)

    s2 = _make_edge_scatter(D2)(g2, src3s, dst3s)

    out = pl.pallas_call(
        _tc3_body,
        out_shape=_f32(NP, N_CLS),
    )(s2, g2, dis, b2r)

    return out[:N]
